# baseline (speedup 1.0000x reference)
import functools

import jax
import jax.numpy as jnp
from jax import lax
from jax.experimental import pallas as pl
from jax.experimental.pallas import tpu as pltpu
from jax.experimental.pallas import tpu_sc as plsc

BATCH = 16384
EMBED_DIM = 64
HIDDEN_DIM = 256
TABLE_ROWS = 1000000
ROWS_PER_TILE = 8
TABLE_TILES = TABLE_ROWS // ROWS_PER_TILE

NUM_CORES = 2
NUM_SUBCORES = 16
NUM_WORKERS = NUM_CORES * NUM_SUBCORES
B_PER_W = BATCH // NUM_WORKERS
CHUNK = 32
N_CHUNKS = B_PER_W // CHUNK
LANES = 16

MLP_TILE = 1024


def _gather_body(user_table, item_table, uid, iid, u_out, i_out,
                 uids_vmem, iids_vmem, sem):
    wid = lax.axis_index("s") * NUM_CORES + lax.axis_index("c")
    base = pl.multiple_of(wid * B_PER_W, B_PER_W)
    pltpu.sync_copy(uid.at[pl.ds(base, B_PER_W)], uids_vmem)
    pltpu.sync_copy(iid.at[pl.ds(base, B_PER_W)], iids_vmem)

    def chunk_body(g, carry):
        off = pl.multiple_of(g * LANES, LANES)
        vu = uids_vmem[pl.ds(off, LANES)]
        vi = iids_vmem[pl.ds(off, LANES)]
        copies = []
        for jj in range(LANES):
            su = vu[jj]
            si = vi[jj]
            copies.append(pltpu.async_copy(
                user_table.at[pl.ds(su, 1)],
                u_out.at[pl.ds(base + off + jj, 1)], sem))
            copies.append(pltpu.async_copy(
                item_table.at[pl.ds(si, 1)],
                i_out.at[pl.ds(base + off + jj, 1)], sem))
        for c in copies:
            c.wait()
        return carry

    lax.fori_loop(0, B_PER_W // LANES, chunk_body, 0)


def _sc_gather(user_table, item_table, user_id, item_id):
    emb = jax.ShapeDtypeStruct((BATCH, EMBED_DIM), jnp.float32)
    fn = functools.partial(
        pl.kernel,
        mesh=plsc.VectorSubcoreMesh(core_axis_name="c", subcore_axis_name="s"),
        out_type=(emb, emb),
        scratch_types=[
            pltpu.VMEM((B_PER_W,), jnp.int32),
            pltpu.VMEM((B_PER_W,), jnp.int32),
            pltpu.SemaphoreType.DMA,
        ],
    )(_gather_body)
    return fn(user_table, item_table, user_id, item_id)


def _mlp_body(u_ref, i_ref, xf_ref, w1u_ref, w1i_ref, w1f_ref, b1_ref,
              w2_ref, b2_ref, o_ref):
    h = jnp.dot(u_ref[...], w1u_ref[...], preferred_element_type=jnp.float32)
    h = h + jnp.dot(i_ref[...], w1i_ref[...], preferred_element_type=jnp.float32)
    h = h + jnp.dot(xf_ref[...], w1f_ref[...], preferred_element_type=jnp.float32)
    h = jnp.maximum(h + b1_ref[...], 0.0)
    y = jnp.dot(h, w2_ref[...], preferred_element_type=jnp.float32) + b2_ref[...]
    o_ref[...] = jax.nn.sigmoid(y)


def _tc_mlp(u_emb, i_emb, xf, W1, b1, W2, b2):
    w1u = W1[:EMBED_DIM]
    w1i = W1[EMBED_DIM:2 * EMBED_DIM]
    w1f = W1[2 * EMBED_DIM:]
    b1_2d = b1.reshape(1, HIDDEN_DIM)
    b2_2d = b2.reshape(1, 1)
    grid = BATCH // MLP_TILE
    out = pl.pallas_call(
        _mlp_body,
        grid=(grid,),
        in_specs=[
            pl.BlockSpec((MLP_TILE, EMBED_DIM), lambda t: (t, 0)),
            pl.BlockSpec((MLP_TILE, EMBED_DIM), lambda t: (t, 0)),
            pl.BlockSpec((MLP_TILE, 2), lambda t: (t, 0)),
            pl.BlockSpec((EMBED_DIM, HIDDEN_DIM), lambda t: (0, 0)),
            pl.BlockSpec((EMBED_DIM, HIDDEN_DIM), lambda t: (0, 0)),
            pl.BlockSpec((2, HIDDEN_DIM), lambda t: (0, 0)),
            pl.BlockSpec((1, HIDDEN_DIM), lambda t: (0, 0)),
            pl.BlockSpec((HIDDEN_DIM, 1), lambda t: (0, 0)),
            pl.BlockSpec((1, 1), lambda t: (0, 0)),
        ],
        out_specs=pl.BlockSpec((MLP_TILE, 1), lambda t: (t, 0)),
        out_shape=jax.ShapeDtypeStruct((BATCH, 1), jnp.float32),
    )(u_emb, i_emb, xf, w1u, w1i, w1f, b1_2d, W2, b2_2d)
    return out[:, 0]


def kernel(user_id, item_id, user_feature, item_feature, user_table,
           item_table, W1, b1, W2, b2):
    u_emb, i_emb = _sc_gather(user_table, item_table, user_id, item_id)
    xf = jnp.stack([user_feature, item_feature], axis=1)
    return _tc_mlp(u_emb, i_emb, xf, W1, b1, W2, b2)
